# baseline (speedup 1.0000x reference)
import dataclasses

import jax
import jax.numpy as jnp
from jax import lax
from jax.experimental import pallas as pl
from jax.experimental.pallas import tpu as pltpu
from jax.experimental.pallas import tpu_sc as plsc

_NUM_INPUTS = 128
_NUM_OUTPUTS = 100000
_BATCH = 4096
_MAX_MOVES = 200

_LANES = 16
_NC = 2
_NS = 16
_NW = _NC * _NS
_ROWS_PER_W = _BATCH // _NW
_MPAD = 208
_CHUNK_A = 112
_CHUNK_B = _MAX_MOVES - _CHUNK_A
_CHUNK_B_PAD = _MPAD - _CHUNK_A
_NWORDS = _NUM_INPUTS // 4
_NKW = _NWORDS // _LANES


def _compiler_params():
    cp = pltpu.CompilerParams(use_tc_tiling_on_sc=False)
    if "needs_layout_passes" in pltpu.CompilerParams.__dataclass_fields__:
        cp = dataclasses.replace(cp, needs_layout_passes=False)
    return cp


def _sc_body(wtab_hbm, hperm_hbm, idx_hbm, bias_hbm, out_hbm,
             idx_v, hid_v, bias_v,
             buf_a0, buf_b0, buf_a1, buf_b1, outrow0, outrow1,
             sem_a0, sem_b0, sem_a1, sem_b1, sem_o0, sem_o1):
    wid = lax.axis_index("s") * _NC + lax.axis_index("c")
    base = wid * _ROWS_PER_W

    pltpu.sync_copy(idx_hbm.at[pl.ds(base, _ROWS_PER_W)], idx_v)
    pltpu.sync_copy(hperm_hbm.at[pl.ds(base, _ROWS_PER_W)], hid_v)
    pltpu.sync_copy(bias_hbm, bias_v)

    lane = lax.iota(jnp.int32, _LANES)
    himask = jnp.full((_LANES,), -65536, jnp.int32)
    shl16 = jnp.full((_LANES,), 16, jnp.int32)
    shl8 = jnp.full((_LANES,), 8, jnp.int32)
    shl24 = jnp.full((_LANES,), 24, jnp.int32)
    one = jnp.full((_LANES,), 1, jnp.int32)
    maxidx = jnp.full((_LANES,), _NUM_OUTPUTS - 1, jnp.int32)
    zero = jnp.zeros((_LANES,), jnp.int32)

    bufs = ((buf_a0, buf_b0), (buf_a1, buf_b1))
    sems = ((sem_a0, sem_b0), (sem_a1, sem_b1))
    outrows = (outrow0, outrow1)
    osems = (sem_o0, sem_o1)

    def issue(row, which, buf, sem):
        col0 = (0, _CHUNK_A)[which]
        size = (_CHUNK_A, _CHUNK_B)[which]
        idx_slice = idx_v.at[row, pl.ds(col0, size)]
        pltpu.async_copy(wtab_hbm.at[idx_slice], buf.at[pl.ds(0, size)], sem)

    def wait(which, buf, sem):
        size = (_CHUNK_A, _CHUNK_B)[which]
        pltpu.make_async_copy(
            wtab_hbm.at[pl.ds(0, size)], buf.at[pl.ds(0, size)], sem).wait()

    def compute(row, which, buf, orow):
        col0 = (0, _CHUNK_A)[which]
        csize = (_CHUNK_A, _CHUNK_B_PAD)[which]
        h = [hid_v[row, pl.ds(k * _LANES, _LANES)] for k in range(4 * _NKW)]

        @pl.loop(0, csize, step=_LANES)
        def _(m0):
            outv = jnp.zeros((_LANES,), jnp.float32)
            for j in range(_LANES):
                m = m0 + j
                acc = jnp.zeros((_LANES,), jnp.float32)
                for k in range(_NKW):
                    w = buf[m, pl.ds(k * _LANES, _LANES)]
                    b0 = lax.shift_right_arithmetic(
                        lax.shift_left(w, shl24), shl24)
                    b1 = lax.shift_right_arithmetic(
                        lax.shift_left(w, shl16), shl24)
                    b2 = lax.shift_right_arithmetic(
                        lax.shift_left(w, shl8), shl24)
                    b3 = lax.shift_right_arithmetic(w, shl24)
                    acc = acc + b0.astype(jnp.float32) * h[4 * k]
                    acc = acc + b1.astype(jnp.float32) * h[4 * k + 1]
                    acc = acc + b2.astype(jnp.float32) * h[4 * k + 2]
                    acc = acc + b3.astype(jnp.float32) * h[4 * k + 3]
                outv = jnp.where(lane == j, jnp.sum(acc), outv)
            bidx = idx_v[row, pl.ds(col0 + m0, _LANES)]
            bidx = jnp.minimum(jnp.maximum(bidx, zero), maxidx)
            pair = plsc.load_gather(
                bias_v, [lax.shift_right_logical(bidx, one)])
            odd = lax.bitwise_and(bidx, one) == one
            bval = plsc.bitcast(
                jnp.where(odd, lax.bitwise_and(pair, himask),
                          lax.shift_left(pair, shl16)), jnp.float32)
            orow[0, pl.ds(col0 + m0, _LANES)] = outv + bval

    issue(0, 0, buf_a0, sem_a0)
    issue(0, 1, buf_b0, sem_b0)
    issue(1, 0, buf_a1, sem_a1)
    issue(1, 1, buf_b1, sem_b1)

    @pl.loop(0, _ROWS_PER_W, step=2)
    def _(row0):
        for d in range(2):
            row = row0 + d
            orow = outrows[d]
            osem = osems[d]

            @pl.when(row >= 2)
            def _():
                pltpu.make_async_copy(
                    orow, out_hbm.at[pl.ds(base, 1)], osem).wait()

            for which in range(2):
                wait(which, bufs[d][which], sems[d][which])
                compute(row, which, bufs[d][which], orow)

                @pl.when(row + 2 < _ROWS_PER_W)
                def _():
                    issue(row + 2, which, bufs[d][which], sems[d][which])

            pltpu.async_copy(orow, out_hbm.at[pl.ds(base + row, 1)], osem)

    for d in range(2):
        pltpu.make_async_copy(
            outrows[d], out_hbm.at[pl.ds(base, 1)], osems[d]).wait()


@jax.jit
def _hidden_to_logits(hidden_layer, legal_moves_idxs, weight, bias):
    amax = jnp.maximum(jnp.max(jnp.abs(weight)), 1e-30)
    scale = amax / 127.0
    q = jnp.round(weight / scale).astype(jnp.int8)
    wtab = lax.bitcast_convert_type(
        q.reshape(_NUM_OUTPUTS, _NWORDS, 4), jnp.int32)
    hperm = ((hidden_layer * scale)
             .reshape(_BATCH, _NKW, _LANES, 4)
             .transpose(0, 1, 3, 2)
             .reshape(_BATCH, _NUM_INPUTS))
    bias_packed = lax.bitcast_convert_type(
        bias.astype(jnp.bfloat16).reshape(_NUM_OUTPUTS // 2, 2), jnp.int32)

    kfn = pl.kernel(
        _sc_body,
        out_type=jax.ShapeDtypeStruct((_BATCH, _MPAD), jnp.float32),
        mesh=plsc.VectorSubcoreMesh(core_axis_name="c", subcore_axis_name="s"),
        compiler_params=_compiler_params(),
        scratch_types=[
            pltpu.VMEM((_ROWS_PER_W, _MAX_MOVES), jnp.int32),
            pltpu.VMEM((_ROWS_PER_W, _NUM_INPUTS), jnp.float32),
            pltpu.VMEM((_NUM_OUTPUTS // 2,), jnp.int32),
            pltpu.VMEM((_CHUNK_A, _NWORDS), jnp.int32),
            pltpu.VMEM((_CHUNK_B_PAD, _NWORDS), jnp.int32),
            pltpu.VMEM((_CHUNK_A, _NWORDS), jnp.int32),
            pltpu.VMEM((_CHUNK_B_PAD, _NWORDS), jnp.int32),
            pltpu.VMEM((1, _MPAD), jnp.float32),
            pltpu.VMEM((1, _MPAD), jnp.float32),
        ] + [pltpu.SemaphoreType.DMA] * 6,
    )
    out = kfn(wtab, hperm, legal_moves_idxs, bias_packed)
    return out[:, :_MAX_MOVES]


def kernel(hidden_layer, legal_moves_idxs, weight, bias):
    return _hidden_to_logits(hidden_layer, legal_moves_idxs, weight, bias)


# speedup vs baseline: 1.4638x; 1.4638x over previous
import dataclasses

import jax
import jax.numpy as jnp
from jax import lax
from jax.experimental import pallas as pl
from jax.experimental.pallas import tpu as pltpu
from jax.experimental.pallas import tpu_sc as plsc

_NUM_INPUTS = 128
_NUM_OUTPUTS = 100000
_BATCH = 4096
_MAX_MOVES = 200

_LANES = 16
_NC = 2
_NS = 16
_NW = _NC * _NS
_ROWS_PER_W = _BATCH // _NW
_MPAD = 208
_CHUNK_A = 112
_CHUNK_B = _MAX_MOVES - _CHUNK_A
_CHUNK_B_PAD = _MPAD - _CHUNK_A
_DCOLS = _NUM_INPUTS + _LANES
_NK = _NUM_INPUTS // _LANES


def _compiler_params():
    cp = pltpu.CompilerParams(use_tc_tiling_on_sc=False)
    if "needs_layout_passes" in pltpu.CompilerParams.__dataclass_fields__:
        cp = dataclasses.replace(cp, needs_layout_passes=False)
    return cp


def _sc_body(wtab_hbm, hid_hbm, idx_hbm, out_hbm,
             idx_v, hid_v,
             buf_a0, buf_b0, buf_a1, buf_b1, outrow0, outrow1,
             sem_a0, sem_b0, sem_a1, sem_b1, sem_o0, sem_o1):
    wid = lax.axis_index("s") * _NC + lax.axis_index("c")
    base = wid * _ROWS_PER_W

    pltpu.sync_copy(idx_hbm.at[pl.ds(base, _ROWS_PER_W)], idx_v)
    pltpu.sync_copy(hid_hbm.at[pl.ds(base, _ROWS_PER_W)], hid_v)

    lane = lax.iota(jnp.int32, _LANES)

    bufs = ((buf_a0, buf_b0), (buf_a1, buf_b1))
    sems = ((sem_a0, sem_b0), (sem_a1, sem_b1))
    outrows = (outrow0, outrow1)
    osems = (sem_o0, sem_o1)

    def issue(row, which, buf, sem):
        col0 = (0, _CHUNK_A)[which]
        size = (_CHUNK_A, _CHUNK_B)[which]
        idx_slice = idx_v.at[row, pl.ds(col0, size)]
        pltpu.async_copy(wtab_hbm.at[idx_slice], buf.at[pl.ds(0, size)], sem)

    def wait(which, buf, sem):
        size = (_CHUNK_A, _CHUNK_B)[which]
        pltpu.make_async_copy(
            wtab_hbm.at[pl.ds(0, size)], buf.at[pl.ds(0, size)], sem).wait()

    def compute(row, which, buf, orow):
        col0 = (0, _CHUNK_A)[which]
        csize = (_CHUNK_A, _CHUNK_B_PAD)[which]
        h = [hid_v[row, pl.ds(k * _LANES, _LANES)] for k in range(_NK)]

        @pl.loop(0, csize, step=_LANES)
        def _(m0):
            outv = jnp.zeros((_LANES,), jnp.float32)
            for j in range(_LANES):
                m = m0 + j
                acc = buf[m, pl.ds(_NUM_INPUTS, _LANES)]
                for k in range(_NK):
                    acc = acc + buf[m, pl.ds(k * _LANES, _LANES)] * h[k]
                outv = jnp.where(lane == j, jnp.sum(acc), outv)
            orow[0, pl.ds(col0 + m0, _LANES)] = outv

    issue(0, 0, buf_a0, sem_a0)
    issue(0, 1, buf_b0, sem_b0)
    issue(1, 0, buf_a1, sem_a1)
    issue(1, 1, buf_b1, sem_b1)

    @pl.loop(0, _ROWS_PER_W, step=2)
    def _(row0):
        for d in range(2):
            row = row0 + d
            orow = outrows[d]
            osem = osems[d]

            @pl.when(row >= 2)
            def _():
                pltpu.make_async_copy(
                    orow, out_hbm.at[pl.ds(base, 1)], osem).wait()

            for which in range(2):
                wait(which, bufs[d][which], sems[d][which])
                compute(row, which, bufs[d][which], orow)

                @pl.when(row + 2 < _ROWS_PER_W)
                def _():
                    issue(row + 2, which, bufs[d][which], sems[d][which])

            pltpu.async_copy(orow, out_hbm.at[pl.ds(base + row, 1)], osem)

    for d in range(2):
        pltpu.make_async_copy(
            outrows[d], out_hbm.at[pl.ds(base, 1)], osems[d]).wait()


@jax.jit
def _hidden_to_logits(hidden_layer, legal_moves_idxs, weight, bias):
    wtab = jnp.concatenate(
        [weight, bias[:, None],
         jnp.zeros((_NUM_OUTPUTS, _LANES - 1), jnp.float32)], axis=1)

    kfn = pl.kernel(
        _sc_body,
        out_type=jax.ShapeDtypeStruct((_BATCH, _MPAD), jnp.float32),
        mesh=plsc.VectorSubcoreMesh(core_axis_name="c", subcore_axis_name="s"),
        compiler_params=_compiler_params(),
        scratch_types=[
            pltpu.VMEM((_ROWS_PER_W, _MAX_MOVES), jnp.int32),
            pltpu.VMEM((_ROWS_PER_W, _NUM_INPUTS), jnp.float32),
            pltpu.VMEM((_CHUNK_A, _DCOLS), jnp.float32),
            pltpu.VMEM((_CHUNK_B_PAD, _DCOLS), jnp.float32),
            pltpu.VMEM((_CHUNK_A, _DCOLS), jnp.float32),
            pltpu.VMEM((_CHUNK_B_PAD, _DCOLS), jnp.float32),
            pltpu.VMEM((1, _MPAD), jnp.float32),
            pltpu.VMEM((1, _MPAD), jnp.float32),
        ] + [pltpu.SemaphoreType.DMA] * 6,
    )
    out = kfn(wtab, hidden_layer, legal_moves_idxs)
    return out[:, :_MAX_MOVES]


def kernel(hidden_layer, legal_moves_idxs, weight, bias):
    return _hidden_to_logits(hidden_layer, legal_moves_idxs, weight, bias)


# speedup vs baseline: 2.1652x; 1.4792x over previous
import dataclasses

import jax
import jax.numpy as jnp
from jax import lax
from jax.experimental import pallas as pl
from jax.experimental.pallas import tpu as pltpu
from jax.experimental.pallas import tpu_sc as plsc

_NUM_INPUTS = 128
_NUM_OUTPUTS = 100000
_BATCH = 4096
_MAX_MOVES = 200

_LANES = 16
_NC = 2
_NS = 16
_NW = _NC * _NS
_ROWS_PER_W = _BATCH // _NW
_MPAD = 208
_CHUNK_A = 112
_CHUNK_B = _MAX_MOVES - _CHUNK_A
_CHUNK_B_PAD = _MPAD - _CHUNK_A
_NKW = _NUM_INPUTS // (2 * _LANES)


def _compiler_params():
    cp = pltpu.CompilerParams(use_tc_tiling_on_sc=False)
    if "needs_layout_passes" in pltpu.CompilerParams.__dataclass_fields__:
        cp = dataclasses.replace(cp, needs_layout_passes=False)
    return cp


def _sc_body(wtab_hbm, hperm_hbm, idx_hbm, bias_hbm, out_hbm,
             idx_v, hid_v, bias_v,
             buf_a0, buf_b0, buf_a1, buf_b1, outrow0, outrow1,
             sem_a0, sem_b0, sem_a1, sem_b1, sem_o0, sem_o1):
    wid = lax.axis_index("s") * _NC + lax.axis_index("c")
    base = wid * _ROWS_PER_W

    pltpu.sync_copy(idx_hbm.at[pl.ds(base, _ROWS_PER_W)], idx_v)
    pltpu.sync_copy(hperm_hbm.at[pl.ds(base, _ROWS_PER_W)], hid_v)
    pltpu.sync_copy(bias_hbm, bias_v)

    lane = lax.iota(jnp.int32, _LANES)
    himask = jnp.full((_LANES,), -65536, jnp.int32)
    shl16 = jnp.full((_LANES,), 16, jnp.int32)
    one = jnp.full((_LANES,), 1, jnp.int32)
    maxidx = jnp.full((_LANES,), _NUM_OUTPUTS - 1, jnp.int32)
    zero = jnp.zeros((_LANES,), jnp.int32)

    bufs = ((buf_a0, buf_b0), (buf_a1, buf_b1))
    sems = ((sem_a0, sem_b0), (sem_a1, sem_b1))
    outrows = (outrow0, outrow1)
    osems = (sem_o0, sem_o1)

    def issue(row, which, buf, sem):
        col0 = (0, _CHUNK_A)[which]
        size = (_CHUNK_A, _CHUNK_B)[which]
        idx_slice = idx_v.at[row, pl.ds(col0, size)]
        pltpu.async_copy(wtab_hbm.at[idx_slice], buf.at[pl.ds(0, size)], sem)

    def wait(which, buf, sem):
        size = (_CHUNK_A, _CHUNK_B)[which]
        pltpu.make_async_copy(
            wtab_hbm.at[pl.ds(0, size)], buf.at[pl.ds(0, size)], sem).wait()

    def compute(row, which, buf, orow):
        col0 = (0, _CHUNK_A)[which]
        csize = (_CHUNK_A, _CHUNK_B_PAD)[which]
        h = [hid_v[row, pl.ds(k * _LANES, _LANES)] for k in range(2 * _NKW)]

        @pl.loop(0, csize, step=_LANES)
        def _(m0):
            accs = [None] * _LANES
            for k in range(_NKW):
                for j in range(_LANES):
                    packed = buf[m0 + j, pl.ds(k * 2 * _LANES, 2 * _LANES)]
                    ci = plsc.bitcast(packed, jnp.int32)
                    wlo = plsc.bitcast(
                        lax.shift_left(ci, shl16), jnp.float32)
                    whi = plsc.bitcast(packed, jnp.float32)
                    if accs[j] is None:
                        accs[j] = wlo * h[2 * k] + whi * h[2 * k + 1]
                    else:
                        accs[j] = accs[j] + wlo * h[2 * k]
                        accs[j] = accs[j] + whi * h[2 * k + 1]
            outv = jnp.zeros((_LANES,), jnp.float32)
            for j in range(_LANES):
                outv = jnp.where(lane == j, jnp.sum(accs[j]), outv)
            bidx = idx_v[row, pl.ds(col0 + m0, _LANES)]
            bidx = jnp.minimum(jnp.maximum(bidx, zero), maxidx)
            pair = plsc.load_gather(
                bias_v, [lax.shift_right_logical(bidx, one)])
            odd = lax.bitwise_and(bidx, one) == one
            bval = plsc.bitcast(
                jnp.where(odd, lax.bitwise_and(pair, himask),
                          lax.shift_left(pair, shl16)), jnp.float32)
            orow[0, pl.ds(col0 + m0, _LANES)] = outv + bval

    issue(0, 0, buf_a0, sem_a0)
    issue(0, 1, buf_b0, sem_b0)
    issue(1, 0, buf_a1, sem_a1)
    issue(1, 1, buf_b1, sem_b1)

    @pl.loop(0, _ROWS_PER_W, step=2)
    def _(row0):
        for d in range(2):
            row = row0 + d
            orow = outrows[d]
            osem = osems[d]

            @pl.when(row >= 2)
            def _():
                pltpu.make_async_copy(
                    orow, out_hbm.at[pl.ds(base, 1)], osem).wait()

            for which in range(2):
                wait(which, bufs[d][which], sems[d][which])
                compute(row, which, bufs[d][which], orow)

                @pl.when(row + 2 < _ROWS_PER_W)
                def _():
                    issue(row + 2, which, bufs[d][which], sems[d][which])

            pltpu.async_copy(orow, out_hbm.at[pl.ds(base + row, 1)], osem)

    for d in range(2):
        pltpu.make_async_copy(
            outrows[d], out_hbm.at[pl.ds(base, 1)], osems[d]).wait()


@jax.jit
def _hidden_to_logits(hidden_layer, legal_moves_idxs, weight, bias):
    wtab = weight.astype(jnp.bfloat16)
    hperm = (hidden_layer.reshape(_BATCH, _NKW, _LANES, 2)
             .transpose(0, 1, 3, 2)
             .reshape(_BATCH, _NUM_INPUTS))
    bias_packed = lax.bitcast_convert_type(
        bias.astype(jnp.bfloat16).reshape(_NUM_OUTPUTS // 2, 2), jnp.int32)

    kfn = pl.kernel(
        _sc_body,
        out_type=jax.ShapeDtypeStruct((_BATCH, _MPAD), jnp.float32),
        mesh=plsc.VectorSubcoreMesh(core_axis_name="c", subcore_axis_name="s"),
        compiler_params=_compiler_params(),
        scratch_types=[
            pltpu.VMEM((_ROWS_PER_W, _MAX_MOVES), jnp.int32),
            pltpu.VMEM((_ROWS_PER_W, _NUM_INPUTS), jnp.float32),
            pltpu.VMEM((_NUM_OUTPUTS // 2,), jnp.int32),
            pltpu.VMEM((_CHUNK_A, _NUM_INPUTS), jnp.bfloat16),
            pltpu.VMEM((_CHUNK_B_PAD, _NUM_INPUTS), jnp.bfloat16),
            pltpu.VMEM((_CHUNK_A, _NUM_INPUTS), jnp.bfloat16),
            pltpu.VMEM((_CHUNK_B_PAD, _NUM_INPUTS), jnp.bfloat16),
            pltpu.VMEM((1, _MPAD), jnp.float32),
            pltpu.VMEM((1, _MPAD), jnp.float32),
        ] + [pltpu.SemaphoreType.DMA] * 6,
    )
    out = kfn(wtab, hperm, legal_moves_idxs, bias_packed)
    return out[:, :_MAX_MOVES]


def kernel(hidden_layer, legal_moves_idxs, weight, bias):
    return _hidden_to_logits(hidden_layer, legal_moves_idxs, weight, bias)
